# baseline (speedup 1.0000x reference)
import functools

import jax
import jax.numpy as jnp
from jax import lax
from jax.experimental import pallas as pl
from jax.experimental.pallas import tpu as pltpu
from jax.experimental.pallas import tpu_sc as plsc

N = 10000
NPAD = 10240
E = 320000
NC = 2
NS = 16
KB = 80
RPT = NPAD // NS


BR = 1280
GR = NPAD // BR


def _tc_first(x, W, A):
    Din = x.shape[1]

    def body(x_ref, w_ref, a_ref, hs_ref, as_ref, ad_ref, scr):
        h = jnp.dot(x_ref[:], w_ref[:], preferred_element_type=jnp.float32)
        hs_ref[:] = h
        contrib = jnp.dot(h, a_ref[:], preferred_element_type=jnp.float32)
        j = pl.program_id(1)

        @pl.when(j == 0)
        def _():
            scr[:] = contrib

        @pl.when(j != 0)
        def _():
            aa = scr[:] + contrib
            as_ref[:] = aa[:, 0:1]
            ad_ref[:] = aa[:, 1:2]

    return pl.pallas_call(
        body,
        grid=(GR, 2),
        in_specs=[
            pl.BlockSpec((BR, Din), lambda i, j: (i, 0)),
            pl.BlockSpec((Din, 128), lambda i, j: (0, j)),
            pl.BlockSpec((128, 2), lambda i, j: (j, 0)),
        ],
        out_specs=[
            pl.BlockSpec((BR, 128), lambda i, j: (j * GR + i, 0)),
            pl.BlockSpec((BR, 1), lambda i, j: (i, 0)),
            pl.BlockSpec((BR, 1), lambda i, j: (i, 0)),
        ],
        out_shape=[
            jax.ShapeDtypeStruct((2 * NPAD, 128), jnp.float32),
            jax.ShapeDtypeStruct((NPAD, 1), jnp.float32),
            jax.ShapeDtypeStruct((NPAD, 1), jnp.float32),
        ],
        scratch_shapes=[pltpu.VMEM((BR, 2), jnp.float32)],
    )(x, W, A)


def _tc_mid(acc1, den1, b1, W2, A2):
    def body(at_ref, ab_ref, dt_ref, db_ref, b_ref, w_ref, a_ref,
             h2_ref, as_ref, ad_ref):
        hT = at_ref[:] / (dt_ref[:] + jnp.float32(1e-16)) + b_ref[:, :128]
        hB = ab_ref[:] / (db_ref[:] + jnp.float32(1e-16)) + b_ref[:, 128:]
        hT = jnp.maximum(hT, jnp.float32(0.0))
        hB = jnp.maximum(hB, jnp.float32(0.0))
        h2 = (jnp.dot(hT, w_ref[:128], preferred_element_type=jnp.float32)
              + jnp.dot(hB, w_ref[128:], preferred_element_type=jnp.float32))
        h2_ref[:] = h2
        aa = jnp.dot(h2, a_ref[:], preferred_element_type=jnp.float32)
        as_ref[:] = aa[:, 0:1]
        ad_ref[:] = aa[:, 1:2]

    return pl.pallas_call(
        body,
        grid=(GR,),
        in_specs=[
            pl.BlockSpec((BR, 128), lambda i: (i, 0)),
            pl.BlockSpec((BR, 128), lambda i: (GR + i, 0)),
            pl.BlockSpec((BR, 1), lambda i: (i, 0)),
            pl.BlockSpec((BR, 1), lambda i: (GR + i, 0)),
            pl.BlockSpec((1, 256), lambda i: (0, 0)),
            pl.BlockSpec((256, 128), lambda i: (0, 0)),
            pl.BlockSpec((128, 2), lambda i: (0, 0)),
        ],
        out_specs=[
            pl.BlockSpec((BR, 128), lambda i: (i, 0)),
            pl.BlockSpec((BR, 1), lambda i: (i, 0)),
            pl.BlockSpec((BR, 1), lambda i: (i, 0)),
        ],
        out_shape=[
            jax.ShapeDtypeStruct((NPAD, 128), jnp.float32),
            jax.ShapeDtypeStruct((NPAD, 1), jnp.float32),
            jax.ShapeDtypeStruct((NPAD, 1), jnp.float32),
        ],
    )(acc1, acc1, den1, den1, b1, W2, A2)


def _sc_gat(src, dst, asv, adv, hmat, *, column_split, DH):
    if column_split:
        ept = E // NS
    else:
        ept = E // (NC * NS)
    out_type = (jax.ShapeDtypeStruct((2 * NPAD, DH), jnp.float32),
                jax.ShapeDtypeStruct((2 * NPAD,), jnp.float32))
    nb = ept // KB

    mesh = plsc.VectorSubcoreMesh(core_axis_name="c", subcore_axis_name="s")

    scratch = [
        [pltpu.VMEM((KB,), jnp.int32)] * 2,
        [pltpu.VMEM((KB,), jnp.int32)] * 2,
        pltpu.VMEM((NPAD,), jnp.float32),
        pltpu.VMEM((NPAD,), jnp.float32),
        [pltpu.VMEM((KB, DH), jnp.float32)] * 2,
        [pltpu.VMEM((KB,), jnp.float32)] * 2,
        [pltpu.VMEM((KB,), jnp.int32)] * 2,
        [pltpu.VMEM((KB,), jnp.int32)] * 2,
        pltpu.VMEM((RPT,), jnp.float32),
        pltpu.VMEM_SHARED((NPAD, DH), jnp.float32),
        pltpu.VMEM_SHARED((NPAD,), jnp.float32),
        [pltpu.SemaphoreType.DMA] * 2,
        [pltpu.SemaphoreType.DMA] * 2,
        [pltpu.SemaphoreType.DMA] * 2,
        [pltpu.SemaphoreType.DMA] * 2,
    ]

    @functools.partial(
        pl.kernel,
        out_type=out_type,
        mesh=mesh,
        scratch_types=scratch,
        compiler_params=pltpu.CompilerParams(needs_layout_passes=False),
    )
    def k(src_h, dst_h, asv_h, adv_h, *hrest):
        (h_h, out_h, den_out_h, srcb, dstb, asv_v, adv_v, rows,
         pbuf, idxs, idxd, zbuf_v, acc_s, den_s, isem, gsem,
         ssem, dsem) = hrest
        c = lax.axis_index("c")
        t = lax.axis_index("s")
        zi = c * 0
        zeros16f = jnp.full((16,), zi.astype(jnp.float32))
        if column_split:
            ebase = t * ept
        else:
            ebase = c * (E // 2) + t * ept

        pltpu.sync_copy(asv_h, asv_v)
        pltpu.sync_copy(adv_h, adv_v)

        def zb(i, _):
            zbuf_v[pl.ds(i * 16, 16)] = zeros16f
            return 0
        lax.fori_loop(0, RPT // 16, zb, 0)

        def zr(j, _):
            for r in range(DH // 16):
                rows[0][j, pl.ds(r * 16, 16)] = zeros16f
            return 0
        lax.fori_loop(0, KB, zr, 0)

        for bb in range(RPT // KB):
            pltpu.sync_copy(rows[0], acc_s.at[pl.ds(t * RPT + bb * KB, KB)])
        pltpu.sync_copy(zbuf_v, den_s.at[pl.ds(t * RPT, RPT)])
        plsc.subcore_barrier()

        def esl(i):
            return pl.ds(ebase + i * KB, KB)

        def idx_issue(i, q):
            pltpu.async_copy(src_h.at[esl(i)], srcb[q], isem[q])
            pltpu.async_copy(dst_h.at[esl(i)], dstb[q], isem[q])

        def idx_wait(i, q):
            pltpu.make_async_copy(src_h.at[esl(i)], srcb[q], isem[q]).wait()
            pltpu.make_async_copy(dst_h.at[esl(i)], dstb[q], isem[q]).wait()

        def pcompute(q):
            for ii in range(KB // 16):
                s16 = srcb[q][pl.ds(ii * 16, 16)]
                d16 = dstb[q][pl.ds(ii * 16, 16)]
                if column_split:
                    idxs[q][pl.ds(ii * 16, 16)] = s16 + c * NPAD
                else:
                    idxs[q][pl.ds(ii * 16, 16)] = s16
                idxd[q][pl.ds(ii * 16, 16)] = d16
                av = plsc.load_gather(asv_v, [s16])
                bv = plsc.load_gather(adv_v, [d16])
                e = av + bv
                e = jnp.maximum(e, e * jnp.float32(0.2))
                pbuf[q][pl.ds(ii * 16, 16)] = jnp.exp(e)
            pltpu.async_copy(pbuf[q], den_s.at[idxd[q]], dsem[q], add=True)
            pltpu.async_copy(h_h.at[idxs[q]], rows[q], gsem[q])

        def den_wait(q):
            pltpu.make_async_copy(pbuf[q], den_s.at[idxd[q]], dsem[q]).wait()

        def gather_wait(q):
            pltpu.make_async_copy(h_h.at[idxs[q]], rows[q], gsem[q]).wait()

        def scatter_wait(q):
            pltpu.make_async_copy(rows[q], acc_s.at[idxd[q]], ssem[q]).wait()

        def step(i, q):
            @pl.when(i + 2 < nb)
            def _():
                idx_issue(i + 2, q)

            @pl.when(i >= 1)
            def _():
                den_wait(1 - q)
                scatter_wait(1 - q)

            @pl.when(i + 1 < nb)
            def _():
                idx_wait(i + 1, 1 - q)
                pcompute(1 - q)

            gather_wait(q)

            def scale(jj, _):
                for k in range(16):
                    j = jj * 16 + k
                    pe = plsc.load_gather(
                        pbuf[q], [jnp.full((16,), j, jnp.int32)])
                    for r in range(DH // 16):
                        rows[q][j, pl.ds(r * 16, 16)] = (
                            rows[q][j, pl.ds(r * 16, 16)] * pe)
                return 0
            lax.fori_loop(0, KB // 16, scale, 0)
            pltpu.async_copy(rows[q], acc_s.at[idxd[q]], ssem[q], add=True)

        pltpu.sync_copy(src_h.at[esl(0)], srcb[0])
        pltpu.sync_copy(dst_h.at[esl(0)], dstb[0])
        if nb > 1:
            idx_issue(1, 1)
        pcompute(0)

        def pair(g, _):
            step(2 * g, 0)
            step(2 * g + 1, 1)
            return 0
        lax.fori_loop(0, nb // 2, pair, 0)
        if nb % 2:
            step(nb - 1, (nb - 1) % 2)
        den_wait((nb - 1) % 2)
        scatter_wait((nb - 1) % 2)
        plsc.subcore_barrier()

        pltpu.sync_copy(acc_s.at[pl.ds(t * RPT, RPT)],
                        out_h.at[pl.ds(c * NPAD + t * RPT, RPT)])
        pltpu.sync_copy(den_s.at[pl.ds(t * RPT, RPT)],
                        den_out_h.at[pl.ds(c * NPAD + t * RPT, RPT)])

    return k(src, dst, asv, adv, hmat)


def _tc_final(accA, accB, denA, denB, b2):
    NP, Dh = accA.shape
    BR = 1280
    grid = NP // BR

    def body(aa_ref, ab_ref, da_ref, db_ref, b_ref, o_ref):
        den = da_ref[:] + db_ref[:] + jnp.float32(1e-16)
        o = (aa_ref[:] + ab_ref[:]) / den + b_ref[:]
        o_ref[:] = jnp.maximum(o, jnp.float32(0.0))

    return pl.pallas_call(
        body,
        grid=(grid,),
        in_specs=[
            pl.BlockSpec((BR, Dh), lambda i: (i, 0)),
            pl.BlockSpec((BR, Dh), lambda i: (i, 0)),
            pl.BlockSpec((BR, 1), lambda i: (i, 0)),
            pl.BlockSpec((BR, 1), lambda i: (i, 0)),
            pl.BlockSpec((1, Dh), lambda i: (0, 0)),
        ],
        out_specs=pl.BlockSpec((BR, Dh), lambda i: (i, 0)),
        out_shape=jax.ShapeDtypeStruct((NP, Dh), jnp.float32),
    )(accA, accB, denA, denB, b2)


def kernel(x, edge_index, W1, a1_src, a1_dst, b1, W2, a2_src, a2_dst, b2):
    src = edge_index[0]
    dst = edge_index[1]
    x_pad = jnp.pad(x, ((0, NPAD - N), (0, 0)))

    A1 = jnp.stack([a1_src, a1_dst], axis=1)
    A2 = jnp.stack([a2_src, a2_dst], axis=1)

    hstack1, as1, ad1 = _tc_first(x_pad, W1, A1)
    acc1, den1 = _sc_gat(src, dst, as1.reshape(NPAD), ad1.reshape(NPAD),
                         hstack1, column_split=True, DH=128)

    h2, as2, ad2 = _tc_mid(acc1, den1[:, None], b1[None, :], W2, A2)

    acc2, den2 = _sc_gat(src, dst, as2.reshape(NPAD), ad2.reshape(NPAD),
                         h2, column_split=False, DH=128)
    out = _tc_final(acc2[:NPAD], acc2[NPAD:],
                    den2[:NPAD, None], den2[NPAD:, None],
                    b2[None, :])
    return out[:N]


# speedup vs baseline: 1.0115x; 1.0115x over previous
import functools

import jax
import jax.numpy as jnp
from jax import lax
from jax.experimental import pallas as pl
from jax.experimental.pallas import tpu as pltpu
from jax.experimental.pallas import tpu_sc as plsc

N = 10000
NPAD = 10240
E = 320000
NC = 2
NS = 16
KB = 80
RPT = NPAD // NS


BR = 1280
GR = NPAD // BR


def _tc_first(x, W, A):
    Din = x.shape[1]

    def body(x_ref, w_ref, a_ref, hs_ref, as_ref, ad_ref, scr):
        h = jnp.dot(x_ref[:], w_ref[:], preferred_element_type=jnp.float32)
        hs_ref[:] = h
        contrib = jnp.dot(h, a_ref[:], preferred_element_type=jnp.float32)
        j = pl.program_id(1)

        @pl.when(j == 0)
        def _():
            scr[:] = contrib

        @pl.when(j != 0)
        def _():
            aa = scr[:] + contrib
            as_ref[:] = aa[:, 0:1]
            ad_ref[:] = aa[:, 1:2]

    return pl.pallas_call(
        body,
        grid=(GR, 2),
        in_specs=[
            pl.BlockSpec((BR, Din), lambda i, j: (i, 0)),
            pl.BlockSpec((Din, 128), lambda i, j: (0, j)),
            pl.BlockSpec((128, 2), lambda i, j: (j, 0)),
        ],
        out_specs=[
            pl.BlockSpec((BR, 128), lambda i, j: (j * GR + i, 0)),
            pl.BlockSpec((BR, 1), lambda i, j: (i, 0)),
            pl.BlockSpec((BR, 1), lambda i, j: (i, 0)),
        ],
        out_shape=[
            jax.ShapeDtypeStruct((2 * NPAD, 128), jnp.float32),
            jax.ShapeDtypeStruct((NPAD, 1), jnp.float32),
            jax.ShapeDtypeStruct((NPAD, 1), jnp.float32),
        ],
        scratch_shapes=[pltpu.VMEM((BR, 2), jnp.float32)],
    )(x, W, A)


def _tc_mid(acc1, den1, b1, W2, A2):
    def body(at_ref, ab_ref, dt_ref, db_ref, b_ref, w_ref, a_ref,
             h2_ref, as_ref, ad_ref):
        hT = at_ref[:] / (dt_ref[:] + jnp.float32(1e-16)) + b_ref[:, :128]
        hB = ab_ref[:] / (db_ref[:] + jnp.float32(1e-16)) + b_ref[:, 128:]
        hT = jnp.maximum(hT, jnp.float32(0.0))
        hB = jnp.maximum(hB, jnp.float32(0.0))
        h2 = (jnp.dot(hT, w_ref[:128], preferred_element_type=jnp.float32)
              + jnp.dot(hB, w_ref[128:], preferred_element_type=jnp.float32))
        h2_ref[:] = h2
        aa = jnp.dot(h2, a_ref[:], preferred_element_type=jnp.float32)
        as_ref[:] = aa[:, 0:1]
        ad_ref[:] = aa[:, 1:2]

    return pl.pallas_call(
        body,
        grid=(GR,),
        in_specs=[
            pl.BlockSpec((BR, 128), lambda i: (i, 0)),
            pl.BlockSpec((BR, 128), lambda i: (GR + i, 0)),
            pl.BlockSpec((BR, 1), lambda i: (i, 0)),
            pl.BlockSpec((BR, 1), lambda i: (GR + i, 0)),
            pl.BlockSpec((1, 256), lambda i: (0, 0)),
            pl.BlockSpec((256, 128), lambda i: (0, 0)),
            pl.BlockSpec((128, 2), lambda i: (0, 0)),
        ],
        out_specs=[
            pl.BlockSpec((BR, 128), lambda i: (i, 0)),
            pl.BlockSpec((BR, 1), lambda i: (i, 0)),
            pl.BlockSpec((BR, 1), lambda i: (i, 0)),
        ],
        out_shape=[
            jax.ShapeDtypeStruct((NPAD, 128), jnp.float32),
            jax.ShapeDtypeStruct((NPAD, 1), jnp.float32),
            jax.ShapeDtypeStruct((NPAD, 1), jnp.float32),
        ],
    )(acc1, acc1, den1, den1, b1, W2, A2)


def _sc_gat(src, dst, asv, adv, hmat, *, column_split, DH):
    if column_split:
        ept = E // NS
    else:
        ept = E // (NC * NS)
    out_type = (jax.ShapeDtypeStruct((2 * NPAD, DH), jnp.float32),
                jax.ShapeDtypeStruct((2 * NPAD,), jnp.float32))
    nb = ept // KB

    mesh = plsc.VectorSubcoreMesh(core_axis_name="c", subcore_axis_name="s")

    scratch = [
        [pltpu.VMEM((KB,), jnp.int32)] * 2,
        [pltpu.VMEM((KB,), jnp.int32)] * 2,
        pltpu.VMEM((NPAD,), jnp.float32),
        pltpu.VMEM((NPAD,), jnp.float32),
        [pltpu.VMEM((KB, DH), jnp.float32)] * 2,
        [pltpu.VMEM((KB,), jnp.float32)] * 2,
        [pltpu.VMEM((KB,), jnp.int32)] * 2,
        [pltpu.VMEM((KB,), jnp.int32)] * 2,
        pltpu.VMEM((RPT,), jnp.float32),
        pltpu.VMEM_SHARED((NPAD, DH), jnp.float32),
        pltpu.VMEM_SHARED((NPAD,), jnp.float32),
        [pltpu.SemaphoreType.DMA] * 2,
        [pltpu.SemaphoreType.DMA] * 2,
        [pltpu.SemaphoreType.DMA] * 2,
        [pltpu.SemaphoreType.DMA] * 2,
    ]

    @functools.partial(
        pl.kernel,
        out_type=out_type,
        mesh=mesh,
        scratch_types=scratch,
        compiler_params=pltpu.CompilerParams(needs_layout_passes=False),
    )
    def k(src_h, dst_h, asv_h, adv_h, *hrest):
        (h_h, out_h, den_out_h, srcb, dstb, asv_v, adv_v, rows,
         pbuf, idxs, idxd, zbuf_v, acc_s, den_s, isem, gsem,
         ssem, dsem) = hrest
        c = lax.axis_index("c")
        t = lax.axis_index("s")
        zi = c * 0
        zeros16f = jnp.full((16,), zi.astype(jnp.float32))
        if column_split:
            ebase = t * ept
        else:
            ebase = c * (E // 2) + t * ept

        pltpu.sync_copy(asv_h, asv_v)
        pltpu.sync_copy(adv_h, adv_v)

        def zb(i, _):
            zbuf_v[pl.ds(i * 16, 16)] = zeros16f
            return 0
        lax.fori_loop(0, RPT // 16, zb, 0)

        def zr(j, _):
            for r in range(DH // 16):
                rows[0][j, pl.ds(r * 16, 16)] = zeros16f
            return 0
        lax.fori_loop(0, KB, zr, 0)

        for bb in range(RPT // KB):
            pltpu.sync_copy(rows[0], acc_s.at[pl.ds(t * RPT + bb * KB, KB)])
        pltpu.sync_copy(zbuf_v, den_s.at[pl.ds(t * RPT, RPT)])
        plsc.subcore_barrier()

        def esl(i):
            return pl.ds(ebase + i * KB, KB)

        def idx_issue(i, q):
            pltpu.async_copy(src_h.at[esl(i)], srcb[q], isem[q])
            pltpu.async_copy(dst_h.at[esl(i)], dstb[q], isem[q])

        def idx_wait(i, q):
            pltpu.make_async_copy(src_h.at[esl(i)], srcb[q], isem[q]).wait()
            pltpu.make_async_copy(dst_h.at[esl(i)], dstb[q], isem[q]).wait()

        def pcompute(q):
            for ii in range(KB // 16):
                s16 = srcb[q][pl.ds(ii * 16, 16)]
                d16 = dstb[q][pl.ds(ii * 16, 16)]
                if column_split:
                    idxs[q][pl.ds(ii * 16, 16)] = s16 + c * NPAD
                else:
                    idxs[q][pl.ds(ii * 16, 16)] = s16
                idxd[q][pl.ds(ii * 16, 16)] = d16
                av = plsc.load_gather(asv_v, [s16])
                bv = plsc.load_gather(adv_v, [d16])
                e = av + bv
                e = jnp.maximum(e, e * jnp.float32(0.2))
                pbuf[q][pl.ds(ii * 16, 16)] = jnp.exp(e)
            pltpu.async_copy(pbuf[q], den_s.at[idxd[q]], dsem[q], add=True)
            pltpu.async_copy(h_h.at[idxs[q]], rows[q], gsem[q])

        def den_wait(q):
            pltpu.make_async_copy(pbuf[q], den_s.at[idxd[q]], dsem[q]).wait()

        def gather_wait(q):
            pltpu.make_async_copy(h_h.at[idxs[q]], rows[q], gsem[q]).wait()

        def scatter_wait(q):
            pltpu.make_async_copy(rows[q], acc_s.at[idxd[q]], ssem[q]).wait()

        def step(i, q):
            @pl.when(i + 2 < nb)
            def _():
                idx_issue(i + 2, q)

            @pl.when(i >= 1)
            def _():
                den_wait(1 - q)
                scatter_wait(1 - q)

            @pl.when(i + 1 < nb)
            def _():
                idx_wait(i + 1, 1 - q)
                pcompute(1 - q)

            gather_wait(q)

            def scale(jj, _):
                for k in range(8):
                    j = jj * 8 + k
                    pe = plsc.load_gather(
                        pbuf[q], [jnp.full((16,), j, jnp.int32)])
                    for r in range(DH // 16):
                        rows[q][j, pl.ds(r * 16, 16)] = (
                            rows[q][j, pl.ds(r * 16, 16)] * pe)
                return 0
            lax.fori_loop(0, KB // 8, scale, 0)
            pltpu.async_copy(rows[q], acc_s.at[idxd[q]], ssem[q], add=True)

        pltpu.sync_copy(src_h.at[esl(0)], srcb[0])
        pltpu.sync_copy(dst_h.at[esl(0)], dstb[0])
        if nb > 1:
            idx_issue(1, 1)
        pcompute(0)

        def pair(g, _):
            step(2 * g, 0)
            step(2 * g + 1, 1)
            return 0
        lax.fori_loop(0, nb // 2, pair, 0)
        if nb % 2:
            step(nb - 1, (nb - 1) % 2)
        den_wait((nb - 1) % 2)
        scatter_wait((nb - 1) % 2)
        plsc.subcore_barrier()

        pltpu.sync_copy(acc_s.at[pl.ds(t * RPT, RPT)],
                        out_h.at[pl.ds(c * NPAD + t * RPT, RPT)])
        pltpu.sync_copy(den_s.at[pl.ds(t * RPT, RPT)],
                        den_out_h.at[pl.ds(c * NPAD + t * RPT, RPT)])

    return k(src, dst, asv, adv, hmat)


def _tc_final(accA, accB, denA, denB, b2):
    NP, Dh = accA.shape
    BR = 1280
    grid = NP // BR

    def body(aa_ref, ab_ref, da_ref, db_ref, b_ref, o_ref):
        den = da_ref[:] + db_ref[:] + jnp.float32(1e-16)
        o = (aa_ref[:] + ab_ref[:]) / den + b_ref[:]
        o_ref[:] = jnp.maximum(o, jnp.float32(0.0))

    return pl.pallas_call(
        body,
        grid=(grid,),
        in_specs=[
            pl.BlockSpec((BR, Dh), lambda i: (i, 0)),
            pl.BlockSpec((BR, Dh), lambda i: (i, 0)),
            pl.BlockSpec((BR, 1), lambda i: (i, 0)),
            pl.BlockSpec((BR, 1), lambda i: (i, 0)),
            pl.BlockSpec((1, Dh), lambda i: (0, 0)),
        ],
        out_specs=pl.BlockSpec((BR, Dh), lambda i: (i, 0)),
        out_shape=jax.ShapeDtypeStruct((NP, Dh), jnp.float32),
    )(accA, accB, denA, denB, b2)


def kernel(x, edge_index, W1, a1_src, a1_dst, b1, W2, a2_src, a2_dst, b2):
    src = edge_index[0]
    dst = edge_index[1]
    x_pad = jnp.pad(x, ((0, NPAD - N), (0, 0)))

    A1 = jnp.stack([a1_src, a1_dst], axis=1)
    A2 = jnp.stack([a2_src, a2_dst], axis=1)

    hstack1, as1, ad1 = _tc_first(x_pad, W1, A1)
    acc1, den1 = _sc_gat(src, dst, as1.reshape(NPAD), ad1.reshape(NPAD),
                         hstack1, column_split=True, DH=128)

    h2, as2, ad2 = _tc_mid(acc1, den1[:, None], b1[None, :], W2, A2)

    acc2, den2 = _sc_gat(src, dst, as2.reshape(NPAD), ad2.reshape(NPAD),
                         h2, column_split=False, DH=128)
    out = _tc_final(acc2[:NPAD], acc2[NPAD:],
                    den2[:NPAD, None], den2[NPAD:, None],
                    b2[None, :])
    return out[:N]
